# Initial kernel scaffold; baseline (speedup 1.0000x reference)
import jax
import jax.numpy as jnp
from jax.experimental import pallas as pl


def kernel(indices, word_vectors):
    raise NotImplementedError("write your pallas kernel here")


# speedup vs baseline: 1.1051x; 1.1051x over previous
import functools

import jax
import jax.numpy as jnp
from jax import lax
from jax.experimental import pallas as pl
from jax.experimental.pallas import tpu as pltpu
from jax.experimental.pallas import tpu_sc as plsc

_B = 16384 * 50
_D = 32
_NW = 32
_PER_W = _B // _NW
_IDX_MINOR = 128
_N_IDX_ROWS = _PER_W // _IDX_MINOR
_GATHERS_PER_GROUP = 10
_GROUP_ROWS = _GATHERS_PER_GROUP * _IDX_MINOR
_N_GROUPS = _N_IDX_ROWS // _GATHERS_PER_GROUP


def _make_sc_gather():
  mesh = plsc.VectorSubcoreMesh(core_axis_name="c", subcore_axis_name="s")

  @functools.partial(
      pl.kernel,
      mesh=mesh,
      out_type=jax.ShapeDtypeStruct((_B, _D), jnp.float32),
      compiler_params=pltpu.CompilerParams(use_tc_tiling_on_sc=False),
      scratch_types=[
          pltpu.VMEM((_N_IDX_ROWS, _IDX_MINOR), jnp.int32),
          pltpu.VMEM((_GROUP_ROWS, _D), jnp.float32),
          pltpu.SemaphoreType.DMA,
      ],
  )
  def gather_kernel(table_hbm, idx_hbm, out_hbm, idx_v, rows_v, gsem):
    wid = lax.axis_index("s") * 2 + lax.axis_index("c")
    pltpu.sync_copy(idx_hbm.at[wid], idx_v)

    def body(g, carry):
      copies = []
      for j in range(_GATHERS_PER_GROUP):
        row = g * _GATHERS_PER_GROUP + j
        copies.append(
            pltpu.async_copy(
                table_hbm.at[idx_v.at[row]],
                rows_v.at[pl.ds(j * _IDX_MINOR, _IDX_MINOR)],
                gsem,
            ))
      for c in copies:
        c.wait()
      start = wid * _PER_W + g * _GROUP_ROWS
      pltpu.sync_copy(rows_v, out_hbm.at[pl.ds(start, _GROUP_ROWS)])
      return carry

    lax.fori_loop(0, _N_GROUPS, body, 0)

  return gather_kernel


_sc_gather = _make_sc_gather()


def kernel(indices, word_vectors):
  idx = indices.reshape(-1).astype(jnp.int32).reshape(
      _NW, _N_IDX_ROWS, _IDX_MINOR)
  out = _sc_gather(word_vectors, idx)
  return out.reshape(indices.shape + (_D,))


# speedup vs baseline: 1.1110x; 1.0054x over previous
import functools

import jax
import jax.numpy as jnp
from jax import lax
from jax.experimental import pallas as pl
from jax.experimental.pallas import tpu as pltpu
from jax.experimental.pallas import tpu_sc as plsc

_B = 16384 * 50
_D = 32
_NW = 32
_PER_W = _B // _NW
_IDX_MINOR = 128
_N_IDX_ROWS = _PER_W // _IDX_MINOR
_GATHERS_PER_GROUP = 10
_GROUP_ROWS = _GATHERS_PER_GROUP * _IDX_MINOR
_N_GROUPS = _N_IDX_ROWS // _GATHERS_PER_GROUP


def _make_sc_gather():
  mesh = plsc.VectorSubcoreMesh(core_axis_name="c", subcore_axis_name="s")

  @functools.partial(
      pl.kernel,
      mesh=mesh,
      out_type=jax.ShapeDtypeStruct((_B, _D), jnp.float32),
      compiler_params=pltpu.CompilerParams(use_tc_tiling_on_sc=False),
      scratch_types=[
          pltpu.VMEM((_N_IDX_ROWS, _IDX_MINOR), jnp.int32),
          pltpu.VMEM((2 * _GROUP_ROWS, _D), jnp.float32),
          pltpu.SemaphoreType.DMA,
          pltpu.SemaphoreType.DMA((2,)),
      ],
  )
  def gather_kernel(table_hbm, idx_hbm, out_hbm, idx_v, rows_v, gsem, ssem):
    wid = lax.axis_index("s") * 2 + lax.axis_index("c")
    pltpu.sync_copy(idx_hbm.at[wid], idx_v)
    out_base = wid * _PER_W

    def body(g, carry):
      b = g % 2
      off = b * _GROUP_ROWS
      start = out_base + g * _GROUP_ROWS

      @pl.when(g >= 2)
      def _():
        pltpu.make_async_copy(
            rows_v.at[pl.ds(off, _GROUP_ROWS)],
            out_hbm.at[pl.ds(start - 2 * _GROUP_ROWS, _GROUP_ROWS)],
            ssem.at[b],
        ).wait()

      copies = []
      for j in range(_GATHERS_PER_GROUP):
        row = g * _GATHERS_PER_GROUP + j
        copies.append(
            pltpu.async_copy(
                table_hbm.at[idx_v.at[row]],
                rows_v.at[pl.ds(off + j * _IDX_MINOR, _IDX_MINOR)],
                gsem,
            ))
      for c in copies:
        c.wait()
      pltpu.make_async_copy(
          rows_v.at[pl.ds(off, _GROUP_ROWS)],
          out_hbm.at[pl.ds(start, _GROUP_ROWS)],
          ssem.at[b],
      ).start()
      return carry

    lax.fori_loop(0, _N_GROUPS, body, 0)

    for g in (_N_GROUPS - 2, _N_GROUPS - 1):
      b = g % 2
      pltpu.make_async_copy(
          rows_v.at[pl.ds(b * _GROUP_ROWS, _GROUP_ROWS)],
          out_hbm.at[pl.ds(out_base + g * _GROUP_ROWS, _GROUP_ROWS)],
          ssem.at[b],
      ).wait()

  return gather_kernel


_sc_gather = _make_sc_gather()


def kernel(indices, word_vectors):
  idx = indices.reshape(-1).astype(jnp.int32).reshape(
      _NW, _N_IDX_ROWS, _IDX_MINOR)
  out = _sc_gather(word_vectors, idx)
  return out.reshape(indices.shape + (_D,))


# speedup vs baseline: 1.6452x; 1.4808x over previous
import functools

import jax
import jax.numpy as jnp
from jax import lax
from jax.experimental import pallas as pl
from jax.experimental.pallas import tpu as pltpu
from jax.experimental.pallas import tpu_sc as plsc

_NB = 16384
_NS = 50
_D = 32
_V = 1000000
_NW = 32
_BT = _NB // 128
_N_BLOCKS = _NS * _BT
_PER_W = _N_BLOCKS // _NW
_IDX_PER_W = _PER_W * 128


def _make_sc_gather():
  mesh = plsc.VectorSubcoreMesh(core_axis_name="c", subcore_axis_name="s")

  @functools.partial(
      pl.kernel,
      mesh=mesh,
      out_type=jax.ShapeDtypeStruct((_NS, _D // 8, _BT, 8, 128), jnp.float32),
      compiler_params=pltpu.CompilerParams(
          use_tc_tiling_on_sc=False, needs_layout_passes=False),
      scratch_types=[
          pltpu.VMEM((_IDX_PER_W,), jnp.int32),
          pltpu.VMEM((2, 128, _D), jnp.float32),
          pltpu.VMEM((2, _D, 128), jnp.float32),
          pltpu.SemaphoreType.DMA,
          pltpu.SemaphoreType.DMA((2,)),
      ],
  )
  def gather_kernel(table_hbm, idx_hbm, out_hbm, idx_v, rows_v, tiles_v,
                    gsem, ssem):
    wid = lax.axis_index("s") * 2 + lax.axis_index("c")
    pltpu.sync_copy(idx_hbm.at[wid], idx_v)
    k_base = wid * _PER_W

    def gather_desc(g, buf):
      return pltpu.make_async_copy(
          table_hbm.at[idx_v.at[pl.ds(g * 128, 128)]],
          rows_v.at[buf],
          gsem,
      )

    def store_descs(g, buf):
      k = k_base + g
      s = k // _BT
      bt = k % _BT
      return [
          pltpu.make_async_copy(
              tiles_v.at[buf, pl.ds(dt * 8, 8)],
              out_hbm.at[s, dt, bt],
              ssem.at[buf],
          )
          for dt in range(_D // 8)
      ]

    gather_desc(0, 0).start()

    def body(g, carry):
      buf = g % 2
      gather_desc(g, buf).wait()
      @pl.when(g + 1 < _PER_W)
      def _():
        gather_desc(g + 1, 1 - buf).start()
      @pl.when(g >= 2)
      def _():
        for c in store_descs(g - 2, buf):
          c.wait()

      def tbody(r, carry2):
        col = jax.lax.broadcast(r, (16,)).astype(jnp.int32)
        for i in range(8):
          rowsel = lax.iota(jnp.int32, 16) + (16 * i)
          vals = plsc.load_gather(rows_v.at[buf], [rowsel, col])
          tiles_v[buf, r, pl.ds(16 * i, 16)] = vals
        return carry2

      lax.fori_loop(0, _D, tbody, 0)

      for c in store_descs(g, buf):
        c.start()
      return carry

    lax.fori_loop(0, _PER_W, body, 0)

    for g in (_PER_W - 2, _PER_W - 1):
      for c in store_descs(g, g % 2):
        c.wait()

  return gather_kernel


_sc_gather = _make_sc_gather()


def kernel(indices, word_vectors):
  idx_t = indices.astype(jnp.int32).T.reshape(_NW, _IDX_PER_W)
  y5 = _sc_gather(word_vectors, idx_t)
  out = jnp.transpose(y5, (2, 4, 0, 1, 3)).reshape(_NB, _NS, _D)
  return out
